# baseline (speedup 1.0000x reference)
import functools

import jax
import jax.numpy as jnp
from jax import lax
from jax.experimental import pallas as pl
from jax.experimental.pallas import tpu as pltpu
from jax.experimental.pallas import tpu_sc as plsc

NC = 2
NS = 16
NW = NC * NS
LANES = 16


@functools.cache
def _make(B, L, V, D):
    LB = B // 128
    RPU = 256
    UPL = B // RPU
    NU = L * UPL
    K = NU // NW
    assert K % 4 == 0
    mesh = plsc.VectorSubcoreMesh(core_axis_name="c", subcore_axis_name="s")

    @functools.partial(
        pl.kernel,
        out_type=jax.ShapeDtypeStruct((L * D * B // 128, 128), jnp.float32),
        mesh=mesh,
        scratch_types=[
            pltpu.VMEM((L, D), jnp.float32),
        ]
        + [pltpu.VMEM((RPU,), jnp.int32)] * 4
        + [pltpu.VMEM((RPU, 128), jnp.float32)] * 2
        + [pltpu.VMEM((D // 8, 24, 129), jnp.float32)] * 2
        + [pltpu.SemaphoreType.DMA] * 8,
        compiler_params=pltpu.CompilerParams(
            use_tc_tiling_on_sc=False, needs_layout_passes=False
        ),
    )
    def k(x4_hbm, table_hbm, pos_hbm, out_hbm, pos_v, *scr):
        idx = scr[0:4]
        g = scr[4:6]
        ov = scr[6:8]
        isem = scr[8:12]
        gsem = scr[12:14]
        ssem = scr[14:16]

        wid = lax.axis_index("s") * NC + lax.axis_index("c")
        u0 = wid * K
        pltpu.sync_copy(pos_hbm, pos_v)

        ji = lax.iota(jnp.int32, 16)
        joA = ji >> 3
        joB = joA + 2
        rowbase = ji & 7

        def unit_lv(u):
            ug = u0 + u
            return ug >> 4, ug & (UPL - 1)

        def start_idx(u, b4):
            l, v = unit_lv(u)
            for s in range(2):
                off = ((l >> 3) * LB + 2 * v + s) * 1024 + (l & 7) * 128
                pltpu.async_copy(
                    x4_hbm.at[pl.ds(off, 128)],
                    idx[b4].at[pl.ds(s * 128, 128)],
                    isem[b4],
                )

        def wait_idx(b4):
            for _ in range(2):
                pltpu.make_async_copy(
                    x4_hbm.at[pl.ds(0, 128)], idx[b4].at[pl.ds(0, 128)], isem[b4]
                ).wait()

        def start_gather(b4, b2):
            pltpu.async_copy(table_hbm.at[idx[b4]], g[b2], gsem[b2])

        def wait_gather(b4, b2):
            pltpu.make_async_copy(table_hbm.at[idx[b4]], g[b2], gsem[b2]).wait()

        def start_store(u, b2):
            l, v = unit_lv(u)
            for jo in range(D // 8):
                for bol in range(2):
                    row0 = (l * (D // 8) + jo) * (B // 16) + v * 16 + bol * 8
                    pltpu.async_copy(
                        ov[b2].at[jo, pl.ds(bol * 8, 8), pl.ds(0, 128)],
                        out_hbm.at[pl.ds(row0, 8), pl.ds(0, 128)],
                        ssem[b2],
                    )

        def wait_store(b2):
            for _ in range(2 * (D // 8)):
                pltpu.make_async_copy(
                    ov[b2].at[0, pl.ds(0, 8), pl.ds(0, 128)],
                    out_hbm.at[pl.ds(0, 8), pl.ds(0, 128)],
                    ssem[b2],
                ).wait()

        def transpose_add(u, b2):
            l, _ = unit_lv(u)
            posA = pos_v[l, pl.ds(0, LANES)]
            posB = pos_v[l, pl.ds(LANES, LANES)]
            for bol in range(2):
                rows = rowbase + bol * 8

                @plsc.parallel_loop(0, 128, unroll=4)
                def body(bi):
                    r = bol * 128 + bi
                    bv = jnp.full((LANES,), bi, jnp.int32)
                    va = g[b2][r, pl.ds(0, LANES)] + posA
                    vb = g[b2][r, pl.ds(LANES, LANES)] + posB
                    plsc.store_scatter(ov[b2], [joA, rows, bv], va)
                    plsc.store_scatter(ov[b2], [joB, rows, bv], vb)

        def step(u, r, *, first, wait_st, prefetch, issue_gather):
            b4 = r % 4
            b2 = r % 2
            if wait_st:
                wait_store(b2)
            wait_gather(b4, b2)
            if prefetch:
                start_idx(u + 4, b4)
            transpose_add(u, b2)
            start_store(u, b2)
            if issue_gather:
                wait_idx((r + 2) % 4)
                start_gather((r + 2) % 4, b2)

        for p in range(4):
            start_idx(p, p)
        for p in range(2):
            wait_idx(p)
            start_gather(p, p)

        for r in range(4):
            step(r, r, first=True, wait_st=(r >= 2), prefetch=True,
                 issue_gather=True)

        def qbody(q, carry):
            for r in range(4):
                step(q * 4 + r, r, first=False, wait_st=True, prefetch=True,
                     issue_gather=True)
            return carry

        lax.fori_loop(1, K // 4 - 1, qbody, 0)

        for r in range(4):
            u = K - 4 + r
            step(u, r, first=False, wait_st=True, prefetch=False,
                 issue_gather=(r < 2))

        for r in range(2, 4):
            wait_store(r % 2)

    return k


def kernel(x, table, pos):
    B, L = x.shape
    V, D = table.shape
    x4 = x.T.reshape(L // 8, 8, B // 128, 128).transpose(0, 2, 1, 3).reshape(-1)
    t = jnp.pad(table, ((0, 0), (0, 128 - D)))
    out5 = _make(B, L, V, D)(x4, t, pos[:L])
    out = (
        out5.reshape(L, D // 8, B // 128, 8, 128)
        .transpose(2, 4, 0, 1, 3)
        .reshape(B, L, D)
    )
    return out


# speedup vs baseline: 1.1470x; 1.1470x over previous
import functools

import jax
import jax.numpy as jnp
from jax import lax
from jax.experimental import pallas as pl
from jax.experimental.pallas import tpu as pltpu
from jax.experimental.pallas import tpu_sc as plsc

NC = 2
NS = 16
NW = NC * NS
LANES = 16


@functools.cache
def _make(B, L, V, D):
    LB = B // 128
    RPU = 256
    UPL = B // RPU
    NU = L * UPL
    K = NU // NW
    assert K % 4 == 0
    mesh = plsc.VectorSubcoreMesh(core_axis_name="c", subcore_axis_name="s")

    @functools.partial(
        pl.kernel,
        out_type=jax.ShapeDtypeStruct((L * D * B // 128, 128), jnp.float32),
        mesh=mesh,
        scratch_types=[
            pltpu.VMEM((L, D), jnp.float32),
        ]
        + [pltpu.VMEM((RPU,), jnp.int32)] * 4
        + [pltpu.VMEM((RPU, D), jnp.float32)] * 2
        + [pltpu.VMEM((D // 8, 24, 129), jnp.float32)] * 2
        + [pltpu.SemaphoreType.DMA] * 8,
        compiler_params=pltpu.CompilerParams(
            use_tc_tiling_on_sc=False, needs_layout_passes=False
        ),
    )
    def k(x4_hbm, table_hbm, pos_hbm, out_hbm, pos_v, *scr):
        idx = scr[0:4]
        g = scr[4:6]
        ov = scr[6:8]
        isem = scr[8:12]
        gsem = scr[12:14]
        ssem = scr[14:16]

        wid = lax.axis_index("s") * NC + lax.axis_index("c")
        u0 = wid * K
        pltpu.sync_copy(pos_hbm, pos_v)

        ji = lax.iota(jnp.int32, 16)
        joA = ji >> 3
        joB = joA + 2
        rowbase = ji & 7

        def unit_lv(u):
            ug = u0 + u
            return ug >> 4, ug & (UPL - 1)

        def start_idx(u, b4):
            l, v = unit_lv(u)
            for s in range(2):
                off = ((l >> 3) * LB + 2 * v + s) * 1024 + (l & 7) * 128
                pltpu.async_copy(
                    x4_hbm.at[pl.ds(off, 128)],
                    idx[b4].at[pl.ds(s * 128, 128)],
                    isem[b4],
                )

        def wait_idx(b4):
            for _ in range(2):
                pltpu.make_async_copy(
                    x4_hbm.at[pl.ds(0, 128)], idx[b4].at[pl.ds(0, 128)], isem[b4]
                ).wait()

        def start_gather(b4, b2):
            pltpu.async_copy(table_hbm.at[idx[b4]], g[b2], gsem[b2])

        def wait_gather(b4, b2):
            pltpu.make_async_copy(table_hbm.at[idx[b4]], g[b2], gsem[b2]).wait()

        def start_store(u, b2):
            l, v = unit_lv(u)
            for jo in range(D // 8):
                for bol in range(2):
                    row0 = (l * (D // 8) + jo) * (B // 16) + v * 16 + bol * 8
                    pltpu.async_copy(
                        ov[b2].at[jo, pl.ds(bol * 8, 8), pl.ds(0, 128)],
                        out_hbm.at[pl.ds(row0, 8), pl.ds(0, 128)],
                        ssem[b2],
                    )

        def wait_store(b2):
            for _ in range(2 * (D // 8)):
                pltpu.make_async_copy(
                    ov[b2].at[0, pl.ds(0, 8), pl.ds(0, 128)],
                    out_hbm.at[pl.ds(0, 8), pl.ds(0, 128)],
                    ssem[b2],
                ).wait()

        def transpose_add(u, b2):
            l, _ = unit_lv(u)
            posA = pos_v[l, pl.ds(0, LANES)]
            posB = pos_v[l, pl.ds(LANES, LANES)]
            for bol in range(2):
                rows = rowbase + bol * 8

                @plsc.parallel_loop(0, 128, unroll=4)
                def body(bi):
                    r = bol * 128 + bi
                    bv = jnp.full((LANES,), bi, jnp.int32)
                    va = g[b2][r, pl.ds(0, LANES)] + posA
                    vb = g[b2][r, pl.ds(LANES, LANES)] + posB
                    plsc.store_scatter(ov[b2], [joA, rows, bv], va)
                    plsc.store_scatter(ov[b2], [joB, rows, bv], vb)

        def step(u, r, *, first, wait_st, prefetch, issue_gather):
            b4 = r % 4
            b2 = r % 2
            if wait_st:
                wait_store(b2)
            wait_gather(b4, b2)
            if prefetch:
                start_idx(u + 4, b4)
            transpose_add(u, b2)
            start_store(u, b2)
            if issue_gather:
                wait_idx((r + 2) % 4)
                start_gather((r + 2) % 4, b2)

        for p in range(4):
            start_idx(p, p)
        for p in range(2):
            wait_idx(p)
            start_gather(p, p)

        for r in range(4):
            step(r, r, first=True, wait_st=(r >= 2), prefetch=True,
                 issue_gather=True)

        def qbody(q, carry):
            for r in range(4):
                step(q * 4 + r, r, first=False, wait_st=True, prefetch=True,
                     issue_gather=True)
            return carry

        lax.fori_loop(1, K // 4 - 1, qbody, 0)

        for r in range(4):
            u = K - 4 + r
            step(u, r, first=False, wait_st=True, prefetch=False,
                 issue_gather=(r < 2))

        for r in range(2, 4):
            wait_store(r % 2)

    return k


def kernel(x, table, pos):
    B, L = x.shape
    V, D = table.shape
    x4 = x.T.reshape(L // 8, 8, B // 128, 128).transpose(0, 2, 1, 3).reshape(-1)
    out5 = _make(B, L, V, D)(x4, table, pos[:L])
    out = (
        out5.reshape(L, D // 8, B // 128, 8, 128)
        .transpose(2, 4, 0, 1, 3)
        .reshape(B, L, D)
    )
    return out
